# baseline (speedup 1.0000x reference)
import functools

import jax
import jax.numpy as jnp
import numpy as np
from jax import lax
from jax.experimental import pallas as pl
from jax.experimental.pallas import tpu as pltpu
from jax.experimental.pallas import tpu_sc as plsc

N_NODES = 10000
D = 128
N_EDGES = 320000

NC = 2
NS = 16
CH = 125
NCHA = 160
NCHE = 80
NBUF = 2
IBUF = 4
XB = 32
ROWS_PER_TILE = 640
N_PAD = NS * ROWS_PER_TILE

_RSQRT_LUT = np.arange(N_EDGES + 2, dtype=np.float32) ** -0.5

_mesh = plsc.VectorSubcoreMesh(core_axis_name="c", subcore_axis_name="s")


@functools.partial(
    pl.kernel,
    out_type=[
        jax.ShapeDtypeStruct((NC, N_PAD, D), jnp.float32),
        jax.ShapeDtypeStruct((NC, N_PAD, D), jnp.float32),
        jax.ShapeDtypeStruct((NC, N_PAD), jnp.float32),
    ],
    mesh=_mesh,
    scratch_types=[
        pltpu.VMEM((IBUF, CH), jnp.int32),
        pltpu.VMEM((NCHE // 2, CH), jnp.int32),
        [pltpu.VMEM((CH, D), jnp.float32) for _ in range(NBUF)],
        [pltpu.VMEM((XB, D), jnp.float32) for _ in range(2)],
        pltpu.VMEM((ROWS_PER_TILE,), jnp.float32),
        pltpu.VMEM((ROWS_PER_TILE,), jnp.int32),
        pltpu.VMEM((128,), jnp.float32),
        pltpu.VMEM_SHARED((N_PAD,), jnp.float32),
        pltpu.VMEM_SHARED((N_PAD, D), jnp.float32),
        [pltpu.SemaphoreType.DMA for _ in range(NBUF)],
        [pltpu.SemaphoreType.DMA for _ in range(IBUF)],
        [pltpu.SemaphoreType.DMA for _ in range(2)],
    ],
)
def _gcn_sc_kernel(row_hbm, col_hbm, x_hbm, lut_hbm,
                   agg_out, xs_out, dis_out,
                   row_v, col_v, rows_v, xblk_v, dis_v, didx_v, ones_v,
                   deg_sh, agg_sh, gsems, rsems, xsems):
    cid = lax.axis_index("c")
    sid = lax.axis_index("s")
    my_rows = pl.ds(sid * ROWS_PER_TILE, ROWS_PER_TILE)

    def zstore(i, _):
        dis_v[pl.ds(i * 16, 16)] = jnp.zeros((16,), jnp.float32)
        return 0

    lax.fori_loop(0, ROWS_PER_TILE // 16, zstore, 0)
    for i in range(8):
        ones_v[pl.ds(i * 16, 16)] = jnp.ones((16,), jnp.float32)
    for r in range(16):
        for k in range(D // 16):
            rows_v[0][r, pl.ds(k * 16, 16)] = jnp.zeros((16,), jnp.float32)
    pltpu.sync_copy(dis_v, deg_sh.at[my_rows])

    def zcopy(k, _):
        pltpu.sync_copy(
            rows_v[0].at[pl.ds(0, 16), :],
            agg_sh.at[pl.ds(sid * ROWS_PER_TILE + k * 16, 16), :])
        return 0

    lax.fori_loop(0, ROWS_PER_TILE // 16, zcopy, 0)
    plsc.subcore_barrier()

    for q in range(IBUF):
        pltpu.async_copy(row_hbm.at[sid, q], row_v.at[q], rsems[q])

    def dbody(jj, _):
        for s in range(IBUF):
            j = jj * IBUF + s
            pltpu.make_async_copy(row_hbm.at[sid, j], row_v.at[s],
                                  rsems[s]).wait()
            pltpu.sync_copy(ones_v.at[pl.ds(0, CH)],
                            deg_sh.at[row_v.at[s]], add=True)

            @pl.when(j + IBUF < NCHA)
            def _():
                pltpu.async_copy(row_hbm.at[sid, j + IBUF], row_v.at[s],
                                 rsems[s])

        return 0

    lax.fori_loop(0, NCHA // IBUF, dbody, 0)
    plsc.subcore_barrier()

    pltpu.sync_copy(deg_sh.at[my_rows], dis_v)

    def cvt(g, _):
        didx_v[pl.ds(g * 16, 16)] = (
            dis_v[pl.ds(g * 16, 16)] + 1.0).astype(jnp.int32)
        return 0

    lax.fori_loop(0, ROWS_PER_TILE // 16, cvt, 0)
    for k in range(ROWS_PER_TILE // 128):
        pltpu.sync_copy(lut_hbm.at[didx_v.at[pl.ds(k * 128, 128)]],
                        dis_v.at[pl.ds(k * 128, 128)])
    pltpu.sync_copy(dis_v, dis_out.at[cid, my_rows])

    for p in range(2):
        pltpu.async_copy(
            x_hbm.at[pl.ds(sid * ROWS_PER_TILE + p * XB, XB), :],
            xblk_v[p], xsems[p])

    def xblock(g, _):
        for p in range(2):
            blk = g * 2 + p
            base = sid * ROWS_PER_TILE + blk * XB
            pltpu.make_async_copy(x_hbm.at[pl.ds(base, XB), :],
                                  xblk_v[p], xsems[p]).wait()

            def sgroup(u, _):
                w16 = dis_v[pl.ds(blk * XB + u * 16, 16)]
                for r in range(16):
                    w = w16[r]
                    for f8 in range(D // 16):
                        xblk_v[p][u * 16 + r, pl.ds(f8 * 16, 16)] = (
                            xblk_v[p][u * 16 + r, pl.ds(f8 * 16, 16)] * w)
                return 0

            lax.fori_loop(0, XB // 16, sgroup, 0)
            pltpu.sync_copy(xblk_v[p], xs_out.at[cid, pl.ds(base, XB), :])

            @pl.when(blk + 2 < ROWS_PER_TILE // XB)
            def _():
                pltpu.async_copy(
                    x_hbm.at[pl.ds(base + 2 * XB, XB), :], xblk_v[p],
                    xsems[p])

        return 0

    lax.fori_loop(0, ROWS_PER_TILE // XB // 2, xblock, 0)
    pltpu.sync_copy(col_hbm.at[sid, pl.ds(cid * NCHE, NCHE // 2)], col_v)
    plsc.subcore_barrier()

    def agg_loop(xs_src):
        for q in range(IBUF):
            pltpu.async_copy(row_hbm.at[sid, cid * NCHE + q], row_v.at[q],
                             rsems[q])
        for b in range(NBUF):
            pltpu.make_async_copy(row_hbm.at[sid, cid * NCHE + b],
                                  row_v.at[b], rsems[b]).wait()
            pltpu.async_copy(xs_src.at[row_v.at[b]], rows_v[b], gsems[b])

        def body(jj, _):
            for s in range(IBUF):
                j = jj * IBUF + s
                b = s % NBUF
                q = s % IBUF
                pltpu.make_async_copy(xs_src.at[row_v.at[q]], rows_v[b],
                                      gsems[b]).wait()

                @pl.when(j == NCHE // 2)
                def _():
                    pltpu.sync_copy(
                        col_hbm.at[sid,
                                   pl.ds(cid * NCHE + NCHE // 2, NCHE // 2)],
                        col_v)

                cj = jnp.where(j < NCHE // 2, j, j - NCHE // 2)
                pltpu.sync_copy(rows_v[b], agg_sh.at[col_v.at[cj]], add=True)

                @pl.when(j + IBUF < NCHE)
                def _():
                    pltpu.async_copy(row_hbm.at[sid, cid * NCHE + j + IBUF],
                                     row_v.at[q], rsems[q])

                qn = (s + NBUF) % IBUF

                @pl.when(j + NBUF < NCHE)
                def _():
                    pltpu.make_async_copy(
                        row_hbm.at[sid, cid * NCHE + j + NBUF],
                        row_v.at[qn], rsems[qn]).wait()
                    pltpu.async_copy(xs_src.at[row_v.at[qn]], rows_v[b],
                                     gsems[b])

            return 0

        lax.fori_loop(0, NCHE // IBUF, body, 0)

    @pl.when(cid == 0)
    def _():
        agg_loop(xs_out.at[0])

    @pl.when(cid == 1)
    def _():
        agg_loop(xs_out.at[1])

    plsc.subcore_barrier()

    pltpu.sync_copy(agg_sh.at[my_rows, :],
                    agg_out.at[cid, my_rows, :])


def _final_body(agg0_ref, agg1_ref, x_ref, dis_ref, wt_ref, out_ref):
    dis = dis_ref[...]
    a = dis * (agg0_ref[...] + agg1_ref[...]) + (dis * dis) * x_ref[...]
    out_ref[...] = jnp.dot(a, wt_ref[...], preferred_element_type=jnp.float32)


def _final_call(agg0, agg1, x, dis, wt):
    nb = N_NODES // 1000
    return pl.pallas_call(
        _final_body,
        grid=(nb,),
        in_specs=[
            pl.BlockSpec((1000, D), lambda i: (i, 0)),
            pl.BlockSpec((1000, D), lambda i: (i, 0)),
            pl.BlockSpec((1000, D), lambda i: (i, 0)),
            pl.BlockSpec((1000, 1), lambda i: (i, 0)),
            pl.BlockSpec((D, D), lambda i: (0, 0)),
        ],
        out_specs=pl.BlockSpec((1000, D), lambda i: (i, 0)),
        out_shape=jax.ShapeDtypeStruct((N_NODES, D), jnp.float32),
    )(agg0, agg1, x, dis, wt)


def kernel(x, edge_index, W):
    ei = edge_index.astype(jnp.int32)
    row = ei[0].reshape(NS, NCHA, CH)
    col = ei[1].reshape(NS, NCHA, CH)
    xp = jnp.concatenate(
        [x, jnp.zeros((N_PAD - N_NODES, D), jnp.float32)], axis=0)
    lut = jnp.asarray(_RSQRT_LUT)
    agg, _, dis = _gcn_sc_kernel(row, col, xp, lut)
    return _final_call(agg[0, :N_NODES], agg[1, :N_NODES], x,
                       dis[0, :N_NODES, None], W.T)


# speedup vs baseline: 1.6582x; 1.6582x over previous
import functools

import jax
import jax.numpy as jnp
from jax import lax
from jax.experimental import pallas as pl
from jax.experimental.pallas import tpu as pltpu
from jax.experimental.pallas import tpu_sc as plsc

N_NODES = 10000
D = 128
N_EDGES = 320000

NC = 2
NS = 16
NW = NC * NS
CH = 125
NCH = 80
E_PAD = NW * NCH * CH
NBUF = 2
IBUF = 4
ROWS_PER_TILE = 640
N_PAD = NS * ROWS_PER_TILE

_mesh = plsc.VectorSubcoreMesh(core_axis_name="c", subcore_axis_name="s")


@functools.partial(
    pl.kernel,
    out_type=jax.ShapeDtypeStruct((NC, N_PAD), jnp.float32),
    mesh=_mesh,
    scratch_types=[
        pltpu.VMEM((NCH, CH), jnp.int32),
        pltpu.VMEM((128,), jnp.float32),
        pltpu.VMEM((ROWS_PER_TILE,), jnp.float32),
        pltpu.VMEM_SHARED((N_PAD,), jnp.float32),
    ],
)
def _deg_kernel(row_hbm, deg_out, idx_v, ones_v, zbuf_v, deg_sh):
    cid = lax.axis_index("c")
    sid = lax.axis_index("s")
    wid = cid * NS + sid

    def zstore(i, _):
        zbuf_v[pl.ds(i * 16, 16)] = jnp.zeros((16,), jnp.float32)
        return 0

    lax.fori_loop(0, ROWS_PER_TILE // 16, zstore, 0)
    for i in range(8):
        ones_v[pl.ds(i * 16, 16)] = jnp.ones((16,), jnp.float32)
    pltpu.sync_copy(zbuf_v, deg_sh.at[pl.ds(sid * ROWS_PER_TILE, ROWS_PER_TILE)])
    pltpu.sync_copy(row_hbm.at[wid], idx_v)
    plsc.subcore_barrier()

    def body(j, _):
        pltpu.sync_copy(ones_v.at[pl.ds(0, CH)], deg_sh.at[idx_v.at[j]],
                        add=True)
        return 0

    lax.fori_loop(0, NCH, body, 0)
    plsc.subcore_barrier()
    pltpu.sync_copy(
        deg_sh.at[pl.ds(sid * ROWS_PER_TILE, ROWS_PER_TILE)],
        deg_out.at[cid, pl.ds(sid * ROWS_PER_TILE, ROWS_PER_TILE)],
    )


@functools.partial(
    pl.kernel,
    out_type=jax.ShapeDtypeStruct((NC, N_PAD, D), jnp.float32),
    mesh=_mesh,
    scratch_types=[
        pltpu.VMEM((IBUF, CH), jnp.int32),
        pltpu.VMEM((NCH, CH), jnp.int32),
        [pltpu.VMEM((CH, D), jnp.float32) for _ in range(NBUF)],
        pltpu.VMEM((16, D), jnp.float32),
        pltpu.VMEM_SHARED((N_PAD, D), jnp.float32),
        [pltpu.SemaphoreType.DMA for _ in range(NBUF)],
        [pltpu.SemaphoreType.DMA for _ in range(IBUF)],
    ],
)
def _agg_kernel(row_hbm, col_hbm, xs_hbm, agg_out,
                row_v, col_v, rows_v, zb_v, agg_sh, gsems, rsems):
    cid = lax.axis_index("c")
    sid = lax.axis_index("s")
    wid = cid * NS + sid

    for r in range(16):
        for k in range(D // 16):
            zb_v[r, pl.ds(k * 16, 16)] = jnp.zeros((16,), jnp.float32)

    def zcopy(k, _):
        pltpu.sync_copy(
            zb_v, agg_sh.at[pl.ds(sid * ROWS_PER_TILE + k * 16, 16), :])
        return 0

    lax.fori_loop(0, ROWS_PER_TILE // 16, zcopy, 0)
    pltpu.sync_copy(col_hbm.at[wid], col_v)
    plsc.subcore_barrier()

    for q in range(IBUF):
        pltpu.async_copy(row_hbm.at[wid, q], row_v.at[q], rsems[q])
    for b in range(NBUF):
        pltpu.make_async_copy(row_hbm.at[wid, b], row_v.at[b],
                              rsems[b]).wait()
        pltpu.async_copy(xs_hbm.at[row_v.at[b]], rows_v[b], gsems[b])

    def body(jj, _):
        for s in range(IBUF):
            j = jj * IBUF + s
            b = s % NBUF
            q = s % IBUF
            pltpu.make_async_copy(xs_hbm.at[row_v.at[q]], rows_v[b],
                                  gsems[b]).wait()
            pltpu.sync_copy(rows_v[b], agg_sh.at[col_v.at[j]], add=True)

            @pl.when(j + IBUF < NCH)
            def _():
                pltpu.async_copy(row_hbm.at[wid, j + IBUF], row_v.at[q],
                                 rsems[q])

            qn = (s + NBUF) % IBUF

            @pl.when(j + NBUF < NCH)
            def _():
                pltpu.make_async_copy(row_hbm.at[wid, j + NBUF],
                                      row_v.at[qn], rsems[qn]).wait()
                pltpu.async_copy(xs_hbm.at[row_v.at[qn]], rows_v[b], gsems[b])

        return 0

    lax.fori_loop(0, NCH // IBUF, body, 0)
    plsc.subcore_barrier()
    pltpu.sync_copy(
        agg_sh.at[pl.ds(sid * ROWS_PER_TILE, ROWS_PER_TILE), :],
        agg_out.at[cid, pl.ds(sid * ROWS_PER_TILE, ROWS_PER_TILE), :],
    )


def _scale_body(deg0_ref, deg1_ref, x_ref, dis_ref, xs_ref):
    d = deg0_ref[...] + deg1_ref[...] + 1.0
    dis = lax.rsqrt(d)
    dis_ref[...] = dis
    xs_ref[...] = dis * x_ref[...]


def _scale_call(deg0, deg1, xp):
    nb = N_PAD // 1024
    return pl.pallas_call(
        _scale_body,
        grid=(nb,),
        in_specs=[
            pl.BlockSpec((1024, 1), lambda i: (i, 0)),
            pl.BlockSpec((1024, 1), lambda i: (i, 0)),
            pl.BlockSpec((1024, D), lambda i: (i, 0)),
        ],
        out_specs=[
            pl.BlockSpec((1024, 1), lambda i: (i, 0)),
            pl.BlockSpec((1024, D), lambda i: (i, 0)),
        ],
        out_shape=[
            jax.ShapeDtypeStruct((N_PAD, 1), jnp.float32),
            jax.ShapeDtypeStruct((N_PAD, D), jnp.float32),
        ],
    )(deg0, deg1, xp)


def _final_body(agg0_ref, agg1_ref, xs_ref, dis_ref, wt_ref, out_ref):
    a = (agg0_ref[...] + agg1_ref[...] + xs_ref[...]) * dis_ref[...]
    out_ref[...] = jnp.dot(a, wt_ref[...], preferred_element_type=jnp.float32)


def _final_call(agg0, agg1, xs, dis, wt):
    nb = N_NODES // 1000
    return pl.pallas_call(
        _final_body,
        grid=(nb,),
        in_specs=[
            pl.BlockSpec((1000, D), lambda i: (i, 0)),
            pl.BlockSpec((1000, D), lambda i: (i, 0)),
            pl.BlockSpec((1000, D), lambda i: (i, 0)),
            pl.BlockSpec((1000, 1), lambda i: (i, 0)),
            pl.BlockSpec((D, D), lambda i: (0, 0)),
        ],
        out_specs=pl.BlockSpec((1000, D), lambda i: (i, 0)),
        out_shape=jax.ShapeDtypeStruct((N_NODES, D), jnp.float32),
    )(agg0, agg1, xs, dis, wt)


def kernel(x, edge_index, W):
    ei = edge_index.astype(jnp.int32)
    pad = jnp.full((E_PAD - N_EDGES,), N_PAD - 1, jnp.int32)
    row = jnp.concatenate([ei[0], pad]).reshape(NW, NCH, CH)
    col = jnp.concatenate([ei[1], pad]).reshape(NW, NCH, CH)
    xp = jnp.concatenate(
        [x, jnp.zeros((N_PAD - N_NODES, D), jnp.float32)], axis=0)
    deg2 = _deg_kernel(row)
    dis, xs = _scale_call(deg2[0, :, None], deg2[1, :, None], xp)
    agg = _agg_kernel(row, col, xs)
    return _final_call(agg[0], agg[1], xs, dis, W.T)


# speedup vs baseline: 1.7186x; 1.0364x over previous
import functools

import jax
import jax.numpy as jnp
from jax import lax
from jax.experimental import pallas as pl
from jax.experimental.pallas import tpu as pltpu
from jax.experimental.pallas import tpu_sc as plsc

N_NODES = 10000
D = 128
N_EDGES = 320000

NC = 2
NS = 16
NW = NC * NS
CH = 125
NCH = 80
E_PAD = NW * NCH * CH
NBUF = 2
IBUF = 4
ROWS_PER_TILE = 640
N_PAD = NS * ROWS_PER_TILE

_mesh = plsc.VectorSubcoreMesh(core_axis_name="c", subcore_axis_name="s")


@functools.partial(
    pl.kernel,
    out_type=jax.ShapeDtypeStruct((NC, N_PAD), jnp.float32),
    mesh=_mesh,
    scratch_types=[
        pltpu.VMEM((NCH, CH), jnp.int32),
        pltpu.VMEM((128,), jnp.float32),
        pltpu.VMEM((ROWS_PER_TILE,), jnp.float32),
        pltpu.VMEM_SHARED((N_PAD,), jnp.float32),
    ],
)
def _deg_kernel(row_hbm, deg_out, idx_v, ones_v, zbuf_v, deg_sh):
    cid = lax.axis_index("c")
    sid = lax.axis_index("s")
    wid = cid * NS + sid

    def zstore(i, _):
        zbuf_v[pl.ds(i * 16, 16)] = jnp.zeros((16,), jnp.float32)
        return 0

    lax.fori_loop(0, ROWS_PER_TILE // 16, zstore, 0)
    for i in range(8):
        ones_v[pl.ds(i * 16, 16)] = jnp.ones((16,), jnp.float32)
    pltpu.sync_copy(zbuf_v, deg_sh.at[pl.ds(sid * ROWS_PER_TILE, ROWS_PER_TILE)])
    pltpu.sync_copy(row_hbm.at[wid], idx_v)
    plsc.subcore_barrier()

    def body(j, _):
        pltpu.sync_copy(ones_v.at[pl.ds(0, CH)], deg_sh.at[idx_v.at[j]],
                        add=True)
        return 0

    lax.fori_loop(0, NCH, body, 0)
    plsc.subcore_barrier()
    pltpu.sync_copy(
        deg_sh.at[pl.ds(sid * ROWS_PER_TILE, ROWS_PER_TILE)],
        deg_out.at[cid, pl.ds(sid * ROWS_PER_TILE, ROWS_PER_TILE)],
    )


@functools.partial(
    pl.kernel,
    out_type=jax.ShapeDtypeStruct((NC, N_PAD, D), jnp.float32),
    mesh=_mesh,
    scratch_types=[
        pltpu.VMEM((IBUF, CH), jnp.int32),
        pltpu.VMEM((NCH, CH), jnp.int32),
        [pltpu.VMEM((CH, D), jnp.float32) for _ in range(NBUF)],
        pltpu.VMEM((16, D), jnp.float32),
        pltpu.VMEM_SHARED((N_PAD, D), jnp.float32),
        [pltpu.SemaphoreType.DMA for _ in range(NBUF)],
        [pltpu.SemaphoreType.DMA for _ in range(IBUF)],
    ],
)
def _agg_kernel(row_hbm, col_hbm, xs_hbm, agg_out,
                row_v, col_v, rows_v, zb_v, agg_sh, gsems, rsems):
    cid = lax.axis_index("c")
    sid = lax.axis_index("s")
    wid = cid * NS + sid

    for r in range(16):
        for k in range(D // 16):
            zb_v[r, pl.ds(k * 16, 16)] = jnp.zeros((16,), jnp.float32)

    def zcopy(k, _):
        pltpu.sync_copy(
            zb_v, agg_sh.at[pl.ds(sid * ROWS_PER_TILE + k * 16, 16), :])
        return 0

    lax.fori_loop(0, ROWS_PER_TILE // 16, zcopy, 0)
    pltpu.sync_copy(col_hbm.at[wid], col_v)
    plsc.subcore_barrier()

    for q in range(IBUF):
        pltpu.async_copy(row_hbm.at[wid, q], row_v.at[q], rsems[q])
    for b in range(NBUF):
        pltpu.make_async_copy(row_hbm.at[wid, b], row_v.at[b],
                              rsems[b]).wait()
        pltpu.async_copy(xs_hbm.at[row_v.at[b]], rows_v[b], gsems[b])

    def body(jj, _):
        for s in range(IBUF):
            j = jj * IBUF + s
            b = s % NBUF
            q = s % IBUF
            pltpu.make_async_copy(xs_hbm.at[row_v.at[q]], rows_v[b],
                                  gsems[b]).wait()
            pltpu.sync_copy(rows_v[b], agg_sh.at[col_v.at[j]], add=True)

            @pl.when(j + IBUF < NCH)
            def _():
                pltpu.async_copy(row_hbm.at[wid, j + IBUF], row_v.at[q],
                                 rsems[q])

            qn = (s + NBUF) % IBUF

            @pl.when(j + NBUF < NCH)
            def _():
                pltpu.make_async_copy(row_hbm.at[wid, j + NBUF],
                                      row_v.at[qn], rsems[qn]).wait()
                pltpu.async_copy(xs_hbm.at[row_v.at[qn]], rows_v[b], gsems[b])

        return 0

    lax.fori_loop(0, NCH // IBUF, body, 0)
    plsc.subcore_barrier()
    pltpu.sync_copy(
        agg_sh.at[pl.ds(sid * ROWS_PER_TILE, ROWS_PER_TILE), :],
        agg_out.at[cid, pl.ds(sid * ROWS_PER_TILE, ROWS_PER_TILE), :],
    )


def _scale_body(deg0_ref, deg1_ref, x_ref, dis_ref, xs_ref):
    d = deg0_ref[...] + deg1_ref[...] + 1.0
    dis = lax.rsqrt(d)
    dis_ref[...] = dis
    xs_ref[...] = dis * x_ref[...]


def _scale_call(deg0, deg1, x):
    nb = N_NODES // 1000
    return pl.pallas_call(
        _scale_body,
        grid=(nb,),
        in_specs=[
            pl.BlockSpec((1000, 1), lambda i: (i, 0)),
            pl.BlockSpec((1000, 1), lambda i: (i, 0)),
            pl.BlockSpec((1000, D), lambda i: (i, 0)),
        ],
        out_specs=[
            pl.BlockSpec((1000, 1), lambda i: (i, 0)),
            pl.BlockSpec((1000, D), lambda i: (i, 0)),
        ],
        out_shape=[
            jax.ShapeDtypeStruct((N_PAD, 1), jnp.float32),
            jax.ShapeDtypeStruct((N_PAD, D), jnp.float32),
        ],
    )(deg0, deg1, x)


def _final_body(agg0_ref, agg1_ref, xs_ref, dis_ref, wt_ref, out_ref):
    a = (agg0_ref[0] + agg1_ref[0] + xs_ref[...]) * dis_ref[...]
    out_ref[...] = jnp.dot(a, wt_ref[...], preferred_element_type=jnp.float32)


def _final_call(agg, xs, dis, wt):
    nb = N_NODES // 1000
    return pl.pallas_call(
        _final_body,
        grid=(nb,),
        in_specs=[
            pl.BlockSpec((1, 1000, D), lambda i: (0, i, 0)),
            pl.BlockSpec((1, 1000, D), lambda i: (1, i, 0)),
            pl.BlockSpec((1000, D), lambda i: (i, 0)),
            pl.BlockSpec((1000, 1), lambda i: (i, 0)),
            pl.BlockSpec((D, D), lambda i: (0, 0)),
        ],
        out_specs=pl.BlockSpec((1000, D), lambda i: (i, 0)),
        out_shape=jax.ShapeDtypeStruct((N_NODES, D), jnp.float32),
    )(agg, agg, xs, dis, wt)


def kernel(x, edge_index, W):
    ei = edge_index.astype(jnp.int32)
    row = ei[0].reshape(NW, NCH, CH)
    col = ei[1].reshape(NW, NCH, CH)
    deg2 = _deg_kernel(row)
    dis, xs = _scale_call(deg2[0, :, None], deg2[1, :, None], x)
    agg = _agg_kernel(row, col, xs)
    return _final_call(agg, xs, dis, W.T)
